# Initial kernel scaffold; baseline (speedup 1.0000x reference)
import jax
import jax.numpy as jnp
from jax.experimental import pallas as pl


def kernel(token_ids, embeddings):
    raise NotImplementedError("write your pallas kernel here")


# speedup vs baseline: 1.1020x; 1.1020x over previous
import functools

import jax
import jax.numpy as jnp
from jax import lax
from jax.experimental import pallas as pl
from jax.experimental.pallas import tpu as pltpu
from jax.experimental.pallas import tpu_sc as plsc

_NUM_TOKENS = 16384 * 50
_DIM = 32

_NC = 2
_NS = 16
_NW = _NC * _NS
_B_PER_W = _NUM_TOKENS // _NW
_CHUNK = 1600
_N_CHUNKS = _B_PER_W // _CHUNK


@functools.partial(
    pl.kernel,
    out_type=jax.ShapeDtypeStruct((_NUM_TOKENS, _DIM), jnp.float32),
    mesh=plsc.VectorSubcoreMesh(core_axis_name="c", subcore_axis_name="s"),
    scratch_types=[
        pltpu.VMEM((_CHUNK,), jnp.int32),
        pltpu.VMEM((_CHUNK, _DIM), jnp.float32),
        pltpu.SemaphoreType.DMA,
    ],
    compiler_params=pltpu.CompilerParams(use_tc_tiling_on_sc=False),
)
def _sc_gather(idx_hbm, table_hbm, out_hbm, idx_v, rows_v, sem):
    wid = lax.axis_index("s") * _NC + lax.axis_index("c")
    base = wid * _B_PER_W

    def body(i, carry):
        off = base + i * _CHUNK
        pltpu.sync_copy(idx_hbm.at[pl.ds(off, _CHUNK)], idx_v)
        pltpu.async_copy(table_hbm.at[idx_v], rows_v, sem).wait()
        pltpu.sync_copy(rows_v, out_hbm.at[pl.ds(off, _CHUNK)])
        return carry

    lax.fori_loop(0, _N_CHUNKS, body, 0)


def kernel(token_ids, embeddings):
    flat_ids = token_ids.reshape(-1)
    out = _sc_gather(flat_ids, embeddings)
    return out.reshape(*token_ids.shape, _DIM)


# speedup vs baseline: 1.1083x; 1.0057x over previous
import functools

import jax
import jax.numpy as jnp
from jax import lax
from jax.experimental import pallas as pl
from jax.experimental.pallas import tpu as pltpu
from jax.experimental.pallas import tpu_sc as plsc

_NUM_TOKENS = 16384 * 50
_DIM = 32

_NC = 2
_NS = 16
_NW = _NC * _NS
_B_PER_W = _NUM_TOKENS // _NW
_CHUNK = 1600
_N_CHUNKS = _B_PER_W // _CHUNK


@functools.partial(
    pl.kernel,
    out_type=jax.ShapeDtypeStruct((_NUM_TOKENS, _DIM), jnp.float32),
    mesh=plsc.VectorSubcoreMesh(core_axis_name="c", subcore_axis_name="s"),
    scratch_types=[
        pltpu.VMEM((2, _CHUNK), jnp.int32),
        pltpu.VMEM((2, _CHUNK, _DIM), jnp.float32),
        pltpu.SemaphoreType.DMA,
        pltpu.SemaphoreType.DMA,
        pltpu.SemaphoreType.DMA,
        pltpu.SemaphoreType.DMA,
    ],
    compiler_params=pltpu.CompilerParams(use_tc_tiling_on_sc=False),
)
def _sc_gather(idx_hbm, table_hbm, out_hbm, idx_v, rows_v, g0, g1, o0, o1):
    wid = lax.axis_index("s") * _NC + lax.axis_index("c")
    base = wid * _B_PER_W
    gsem = (g0, g1)
    osem = (o0, o1)

    def idx_load(i, b):
        pltpu.sync_copy(idx_hbm.at[pl.ds(base + i * _CHUNK, _CHUNK)],
                        idx_v.at[b])

    def gather_start(i, b):
        return pltpu.async_copy(table_hbm.at[idx_v.at[b]], rows_v.at[b],
                                gsem[b])

    def out_start(i, b):
        return pltpu.async_copy(rows_v.at[b],
                                out_hbm.at[pl.ds(base + i * _CHUNK, _CHUNK)],
                                osem[b])

    idx_load(0, 0)
    gathers = {0: gather_start(0, 0)}
    outs = {}
    for i in range(_N_CHUNKS):
        b = i % 2
        b2 = (i + 1) % 2
        if i + 1 < _N_CHUNKS:
            if i >= 1:
                outs.pop(i - 1).wait()
            idx_load(i + 1, b2)
            gathers[i + 1] = gather_start(i + 1, b2)
        gathers.pop(i).wait()
        outs[i] = out_start(i, b)
    outs.pop(_N_CHUNKS - 2).wait()
    outs.pop(_N_CHUNKS - 1).wait()


def kernel(token_ids, embeddings):
    flat_ids = token_ids.reshape(-1)
    out = _sc_gather(flat_ids, embeddings)
    return out.reshape(*token_ids.shape, _DIM)
